# Initial kernel scaffold; baseline (speedup 1.0000x reference)
import jax
import jax.numpy as jnp
from jax.experimental import pallas as pl


def kernel(atom_types, pos, edge_index, edge_offset, triplet_idx, batch, num_atoms, volume, params):
    raise NotImplementedError("write your pallas kernel here")


# speedup vs baseline: 1.1731x; 1.1731x over previous
import functools
import math

import jax
import jax.numpy as jnp
from jax import lax
from jax.experimental import pallas as pl
from jax.experimental.pallas import tpu as pltpu

Nn = 10000
Ee = 320000
Tt = 320000
Bb = 4
F = 64
G = 80
A = 16
NC = 2
CUT = 5.0

BE = 2000
BT = 2000

def _centers_r():
    i = lax.broadcasted_iota(jnp.int32, (1, G), 1).astype(jnp.float32)
    return i * (CUT / (G - 1))


def _centers_a():
    i = lax.broadcasted_iota(jnp.int32, (1, A), 1).astype(jnp.float32)
    return i * (2.0 / (A - 1)) - 1.0


def _softplus(x):
    return jnp.logaddexp(x, 0.0)


def _edge_basis(dist_col):
    d = dist_col - _centers_r()
    return jnp.exp(-(d * d) * 25.0)


def _erec_body(pos_s_ref, pos_d_ref, off_ref, out_ref):
    r = pos_s_ref[...] - pos_d_ref[...] + off_ref[...]
    d2 = jnp.sum(r * r, axis=1, keepdims=True)
    dist = jnp.sqrt(d2 + 1e-12)
    cw = 0.5 * (jnp.cos(dist * (math.pi / CUT)) + 1.0)
    out_ref[...] = jnp.concatenate(
        [r, dist, cw, jnp.zeros_like(r)], axis=1)


def _edge_record(pos_s, pos_d, off):
    grid = Ee // BE
    return pl.pallas_call(
        _erec_body,
        grid=(grid,),
        in_specs=[pl.BlockSpec((BE, 3), lambda i: (i, 0))] * 3,
        out_specs=pl.BlockSpec((BE, 8), lambda i: (i, 0)),
        out_shape=jax.ShapeDtypeStruct((Ee, 8), jnp.float32),
    )(pos_s, pos_d, off)


def _trec_body(er1_ref, er2_ref, out_ref):
    er1 = er1_ref[...]
    er2 = er2_ref[...]
    v1 = er1[:, 0:3]
    v2 = er2[:, 0:3]
    d1 = jnp.maximum(er1[:, 3:4], 1e-8)
    d2 = jnp.maximum(er2[:, 3:4], 1e-8)
    cos = jnp.clip(jnp.sum(v1 * v2, axis=1, keepdims=True) / (d1 * d2),
                   -1.0, 1.0)
    dd = cos - _centers_a()
    ang = jnp.exp(-(dd * dd) * (1.0 / 0.0225))
    cw2 = er1[:, 4:5] * er2[:, 4:5]
    pad = jnp.zeros((er1.shape[0], 5), jnp.float32)
    out_ref[...] = jnp.concatenate([ang, d1, d2, cw2, pad], axis=1)


def _triplet_record(er1, er2):
    grid = Tt // BT
    return pl.pallas_call(
        _trec_body,
        grid=(grid,),
        in_specs=[pl.BlockSpec((BT, 8), lambda i: (i, 0))] * 2,
        out_specs=pl.BlockSpec((BT, 24), lambda i: (i, 0)),
        out_shape=jax.ShapeDtypeStruct((Tt, 24), jnp.float32),
    )(er1, er2)


def _two_t(gd, gs, erec, W_ref, b_ref):
    dist = erec[:, 3:4]
    ef = _edge_basis(dist)
    t = jnp.dot(gd, W_ref[0:F, :], preferred_element_type=jnp.float32)
    t += jnp.dot(gs, W_ref[F:2 * F, :], preferred_element_type=jnp.float32)
    t += jnp.dot(ef, W_ref[2 * F:, :], preferred_element_type=jnp.float32)
    return t + b_ref[...]


def _two_pass1_body(gd_ref, gs_ref, er_ref, W_ref, b_ref, out_ref):
    i = pl.program_id(0)
    t = _two_t(gd_ref[...], gs_ref[...], er_ref[...], W_ref, b_ref)

    @pl.when(i == 0)
    def _():
        out_ref[...] = jnp.zeros_like(out_ref)

    out_ref[0:1, :] += jnp.sum(t, axis=0, keepdims=True)
    out_ref[1:2, :] += jnp.sum(t * t, axis=0, keepdims=True)


def _two_pass2_body(gd_ref, gs_ref, er_ref, W_ref, b_ref, st_ref, g1_ref,
                    be1_ref, out_ref):
    t = _two_t(gd_ref[...], gs_ref[...], er_ref[...], W_ref, b_ref)
    mu = st_ref[0:1, :] * (1.0 / Ee)
    var = st_ref[1:2, :] * (1.0 / Ee) - mu * mu
    tn = (t - mu) / jnp.sqrt(var + 1e-5) * g1_ref[...] + be1_ref[...]
    gate = jax.nn.sigmoid(tn[:, :F])
    core = _softplus(tn[:, F:])
    cw = er_ref[...][:, 4:5]
    out_ref[...] = gate * core * cw


def _two_stage(gd, gs, erec, W, b, g1, be1):
    grid = Ee // BE
    row = lambda i: (i, 0)
    whole = lambda i: (0, 0)
    b2 = b.reshape(1, 2 * F)
    g1r = g1.reshape(1, 2 * F)
    be1r = be1.reshape(1, 2 * F)
    stats = pl.pallas_call(
        _two_pass1_body,
        grid=(grid,),
        in_specs=[pl.BlockSpec((BE, F), row), pl.BlockSpec((BE, F), row),
                  pl.BlockSpec((BE, 8), row),
                  pl.BlockSpec((2 * F + G, 2 * F), whole),
                  pl.BlockSpec((1, 2 * F), whole)],
        out_specs=pl.BlockSpec((8, 2 * F), whole),
        out_shape=jax.ShapeDtypeStruct((8, 2 * F), jnp.float32),
    )(gd, gs, erec, W, b2)
    msg = pl.pallas_call(
        _two_pass2_body,
        grid=(grid,),
        in_specs=[pl.BlockSpec((BE, F), row), pl.BlockSpec((BE, F), row),
                  pl.BlockSpec((BE, 8), row),
                  pl.BlockSpec((2 * F + G, 2 * F), whole),
                  pl.BlockSpec((1, 2 * F), whole),
                  pl.BlockSpec((8, 2 * F), whole),
                  pl.BlockSpec((1, 2 * F), whole),
                  pl.BlockSpec((1, 2 * F), whole)],
        out_specs=pl.BlockSpec((BE, F), row),
        out_shape=jax.ShapeDtypeStruct((Ee, F), jnp.float32),
    )(gd, gs, erec, W, b2, stats, g1r, be1r)
    return msg


def _three_t(gca, trec, Q_ref, b_ref):
    d1 = trec[:, A:A + 1]
    d2 = trec[:, A + 1:A + 2]
    ang = trec[:, 0:A]
    ef1 = _edge_basis(d1)
    ef2 = _edge_basis(d2)
    t = jnp.dot(gca, Q_ref[0:F, :], preferred_element_type=jnp.float32)
    t += jnp.dot(ef1, Q_ref[F:F + G, :], preferred_element_type=jnp.float32)
    t += jnp.dot(ef2, Q_ref[F + G:F + 2 * G, :],
                 preferred_element_type=jnp.float32)
    t += jnp.dot(ang, Q_ref[F + 2 * G:, :], preferred_element_type=jnp.float32)
    return t + b_ref[...]


def _three_pass1_body(gca_ref, tr_ref, Q_ref, b_ref, out_ref):
    i = pl.program_id(0)
    t = _three_t(gca_ref[...], tr_ref[...], Q_ref, b_ref)

    @pl.when(i == 0)
    def _():
        out_ref[...] = jnp.zeros_like(out_ref)

    out_ref[0:1, :] += jnp.sum(t, axis=0, keepdims=True)
    out_ref[1:2, :] += jnp.sum(t * t, axis=0, keepdims=True)


def _three_pass2_body(gca_ref, tr_ref, Q_ref, b_ref, st_ref, g1_ref, be1_ref,
                      out_ref):
    t = _three_t(gca_ref[...], tr_ref[...], Q_ref, b_ref)
    mu = st_ref[0:1, :] * (1.0 / Tt)
    var = st_ref[1:2, :] * (1.0 / Tt) - mu * mu
    tn = (t - mu) / jnp.sqrt(var + 1e-5) * g1_ref[...] + be1_ref[...]
    gate = jax.nn.sigmoid(tn[:, :F])
    core = _softplus(tn[:, F:])
    cw2 = tr_ref[...][:, A + 2:A + 3]
    out_ref[...] = gate * core * cw2


def _three_stage(gca, trec, Q, b, g1, be1):
    grid = Tt // BT
    row = lambda i: (i, 0)
    whole = lambda i: (0, 0)
    b2 = b.reshape(1, 2 * F)
    g1r = g1.reshape(1, 2 * F)
    be1r = be1.reshape(1, 2 * F)
    K = F + 2 * G + A
    stats = pl.pallas_call(
        _three_pass1_body,
        grid=(grid,),
        in_specs=[pl.BlockSpec((BT, F), row), pl.BlockSpec((BT, 24), row),
                  pl.BlockSpec((K, 2 * F), whole),
                  pl.BlockSpec((1, 2 * F), whole)],
        out_specs=pl.BlockSpec((8, 2 * F), whole),
        out_shape=jax.ShapeDtypeStruct((8, 2 * F), jnp.float32),
    )(gca, trec, Q, b2)
    msg = pl.pallas_call(
        _three_pass2_body,
        grid=(grid,),
        in_specs=[pl.BlockSpec((BT, F), row), pl.BlockSpec((BT, 24), row),
                  pl.BlockSpec((K, 2 * F), whole),
                  pl.BlockSpec((1, 2 * F), whole),
                  pl.BlockSpec((8, 2 * F), whole),
                  pl.BlockSpec((1, 2 * F), whole),
                  pl.BlockSpec((1, 2 * F), whole)],
        out_specs=pl.BlockSpec((BT, F), row),
        out_shape=jax.ShapeDtypeStruct((Tt, F), jnp.float32),
    )(gca, trec, Q, b2, stats, g1r, be1r)
    return msg


def _node_body(af_ref, ag_ref, g2_ref, be2_ref, out_ref):
    ag = ag_ref[...]
    mu = jnp.mean(ag, axis=0, keepdims=True)
    var = jnp.mean(ag * ag, axis=0, keepdims=True) - mu * mu
    an = (ag - mu) / jnp.sqrt(var + 1e-5) * g2_ref[...] + be2_ref[...]
    out_ref[...] = _softplus(af_ref[...] + an)


def _node_update(af, aggr, g2, be2):
    return pl.pallas_call(
        _node_body,
        in_specs=[pl.BlockSpec((Nn, F), lambda: (0, 0)),
                  pl.BlockSpec((Nn, F), lambda: (0, 0)),
                  pl.BlockSpec((1, F), lambda: (0, 0)),
                  pl.BlockSpec((1, F), lambda: (0, 0))],
        out_specs=pl.BlockSpec((Nn, F), lambda: (0, 0)),
        out_shape=jax.ShapeDtypeStruct((Nn, F), jnp.float32),
    )(af, aggr, g2.reshape(1, F), be2.reshape(1, F))


def _head_body(af_ref, batch_ref, W1_ref, b1_ref, W2_ref, b2_ref, W3_ref,
               b3_ref, out_ref):
    h = _softplus(jnp.dot(af_ref[...], W1_ref[...],
                          preferred_element_type=jnp.float32) + b1_ref[...])
    h = _softplus(jnp.dot(h, W2_ref[...],
                          preferred_element_type=jnp.float32) + b2_ref[...])
    e = jnp.dot(h, W3_ref[...], preferred_element_type=jnp.float32) \
        + b3_ref[...]
    mask = (batch_ref[...] == lax.broadcasted_iota(jnp.int32, (1, Bb), 1))
    out_ref[...] = jnp.sum(e * mask.astype(jnp.float32), axis=0,
                           keepdims=True)


def _head(af, batch, params_head):
    (W1, b1), (W2, b2), (W3, b3) = params_head
    whole = lambda: (0, 0)
    out = pl.pallas_call(
        _head_body,
        in_specs=[pl.BlockSpec((Nn, F), whole),
                  pl.BlockSpec((Nn, 1), whole),
                  pl.BlockSpec((F, 128), whole),
                  pl.BlockSpec((1, 128), whole),
                  pl.BlockSpec((128, F), whole),
                  pl.BlockSpec((1, F), whole),
                  pl.BlockSpec((F, 1), whole),
                  pl.BlockSpec((1, 1), whole)],
        out_specs=pl.BlockSpec((1, Bb), whole),
        out_shape=jax.ShapeDtypeStruct((1, Bb), jnp.float32),
    )(af, batch.reshape(Nn, 1), W1, b1.reshape(1, 128), W2,
      b2.reshape(1, F), W3, b3.reshape(1, 1))
    return out.reshape(Bb)


def kernel(atom_types, pos, edge_index, edge_offset, triplet_idx, batch,
           num_atoms, volume, params):
    src = edge_index[0]
    dst = edge_index[1]
    e1 = triplet_idx[0]
    e2 = triplet_idx[1]

    erec = _edge_record(pos[src], pos[dst], edge_offset)
    trec = _triplet_record(erec[e1], erec[e2])
    ca = dst[e1]

    af = params['embed'][atom_types]
    for l in range(NC):
        p = params['two'][l]
        msg = _two_stage(af[dst], af[src], erec, p['W'], p['b'], p['g1'],
                         p['be1'])
        aggr = jax.ops.segment_sum(msg, dst, num_segments=Nn)
        af = _node_update(af, aggr, p['g2'], p['be2'])

        q = params['three'][l]
        msg2 = _three_stage(af[ca], trec, q['W'], q['b'], q['g1'], q['be1'])
        aggr2 = jax.ops.segment_sum(msg2, ca, num_segments=Nn)
        af = _node_update(af, aggr2, q['g2'], q['be2'])

    return _head(af, batch, params['head'])


# speedup vs baseline: 1.7970x; 1.5319x over previous
import functools
import math

import jax
import jax.numpy as jnp
from jax import lax
from jax.experimental import pallas as pl
from jax.experimental.pallas import tpu as pltpu
from jax.experimental.pallas import tpu_sc as plsc

Nn = 10000
Ee = 320000
Tt = 320000
Bb = 4
F = 64
G = 80
A = 16
NC = 2
CUT = 5.0

EP = 327680
TP = 327680
BE = 2048
BT = 2048
NW = 32
CH = 128


@functools.lru_cache(None)
def _make_sc_gather(n_dma, D, k):
    M = NW * n_dma * CH
    mesh = plsc.VectorSubcoreMesh(core_axis_name="c", subcore_axis_name="s")

    def body(table_hbm, idx_hbm, out_hbm, idx_v, bufs, semg, semo):
        wid = lax.axis_index("s") * 2 + lax.axis_index("c")
        pltpu.sync_copy(idx_hbm.at[wid], idx_v)
        base = wid * (n_dma * CH)
        out_h = [None, None]
        g = 0
        j0 = 0
        while j0 < n_dma:
            par = g % 2
            kk = min(k, n_dma - j0)
            if out_h[par] is not None:
                out_h[par].wait()
            hs = []
            for i in range(kk):
                hs.append(pltpu.async_copy(
                    table_hbm.at[idx_v.at[j0 + i]],
                    bufs.at[par, pl.ds(i * CH, CH)], semg))
            for h in hs:
                h.wait()
            out_h[par] = pltpu.async_copy(
                bufs.at[par, pl.ds(0, kk * CH)],
                out_hbm.at[pl.ds(base + j0 * CH, kk * CH)], semo)
            g += 1
            j0 += kk
        for h in out_h:
            if h is not None:
                h.wait()

    return pl.kernel(
        body,
        out_type=jax.ShapeDtypeStruct((M, D), jnp.float32),
        mesh=mesh,
        compiler_params=pltpu.CompilerParams(use_tc_tiling_on_sc=False),
        scratch_types=[pltpu.VMEM((n_dma, CH), jnp.int32),
                       pltpu.VMEM((2, k * CH, D), jnp.float32),
                       pltpu.SemaphoreType.DMA,
                       pltpu.SemaphoreType.DMA])


@functools.lru_cache(None)
def _make_sc_scatter_add(n_dma, Nrow, D):
    mesh = plsc.VectorSubcoreMesh(core_axis_name="c", subcore_axis_name="s")
    rps = Nrow // 16

    def body(msg_hbm, idx_hbm, zero_hbm, out_hbm, idx_v, bufs, acc, sem):
        c = lax.axis_index("c")
        s = lax.axis_index("s")
        wid = s * 2 + c
        pltpu.sync_copy(idx_hbm.at[wid], idx_v)
        pltpu.sync_copy(zero_hbm.at[pl.ds(s * rps, rps)],
                        acc.at[pl.ds(s * rps, rps)])
        plsc.subcore_barrier()
        base = wid * (n_dma * CH)
        h = [None, None]
        h[0] = pltpu.async_copy(msg_hbm.at[pl.ds(base, CH)], bufs.at[0], sem)
        for j in range(n_dma):
            par = j % 2
            if j + 1 < n_dma:
                h[1 - par] = pltpu.async_copy(
                    msg_hbm.at[pl.ds(base + (j + 1) * CH, CH)],
                    bufs.at[1 - par], sem)
            h[par].wait()
            pltpu.sync_copy(bufs.at[par], acc.at[idx_v.at[j]], add=True)
        plsc.subcore_barrier()
        pltpu.sync_copy(acc.at[pl.ds(s * rps, rps)],
                        out_hbm.at[c, pl.ds(s * rps, rps)])

    return pl.kernel(
        body,
        out_type=jax.ShapeDtypeStruct((2, Nrow, D), jnp.float32),
        mesh=mesh,
        compiler_params=pltpu.CompilerParams(use_tc_tiling_on_sc=False),
        scratch_types=[pltpu.VMEM((n_dma, CH), jnp.int32),
                       pltpu.VMEM((2, CH, D), jnp.float32),
                       pltpu.VMEM_SHARED((Nrow, D), jnp.float32),
                       pltpu.SemaphoreType.DMA])


def _sc_gather(table, idx3, D, k):
    return _make_sc_gather(idx3.shape[1], D, k)(table, idx3)


def _sc_scatter_add(msg, idx3, zero):
    return _make_sc_scatter_add(idx3.shape[1], Nn, F)(msg, idx3, zero)


def _centers_r():
    i = lax.broadcasted_iota(jnp.int32, (1, G), 1).astype(jnp.float32)
    return i * (CUT / (G - 1))


def _centers_a():
    i = lax.broadcasted_iota(jnp.int32, (1, A), 1).astype(jnp.float32)
    return i * (2.0 / (A - 1)) - 1.0


def _softplus(x):
    return jnp.logaddexp(x, 0.0)


def _edge_basis(dist_col):
    d = dist_col - _centers_r()
    return jnp.exp(-(d * d) * 25.0)


def _row_mask(limit, nrows):
    i = pl.program_id(0)
    rid = lax.broadcasted_iota(jnp.int32, (nrows, 1), 0) + i * nrows
    return (rid < limit).astype(jnp.float32)


def _erec_body(pos_s_ref, pos_d_ref, off_ref, dst_ref, out_ref):
    r = pos_s_ref[...][:, 0:3] - pos_d_ref[...][:, 0:3] + off_ref[...]
    d2 = jnp.sum(r * r, axis=1, keepdims=True)
    dist = jnp.sqrt(d2 + 1e-12)
    cw = 0.5 * (jnp.cos(dist * (math.pi / CUT)) + 1.0)
    cw = cw * _row_mask(Ee, BE)
    dstf = dst_ref[...].astype(jnp.float32)
    out_ref[...] = jnp.concatenate(
        [r, dist, cw, dstf, jnp.zeros((BE, 10), jnp.float32)], axis=1)


def _edge_record(posg, off_p, dst_p):
    grid = EP // BE
    return pl.pallas_call(
        _erec_body,
        grid=(grid,),
        in_specs=[pl.BlockSpec((BE, 16), lambda i: (i, 0)),
                  pl.BlockSpec((BE, 16), lambda i: (i + EP // BE, 0)),
                  pl.BlockSpec((BE, 3), lambda i: (i, 0)),
                  pl.BlockSpec((BE, 1), lambda i: (i, 0))],
        out_specs=pl.BlockSpec((BE, 16), lambda i: (i, 0)),
        out_shape=jax.ShapeDtypeStruct((EP, 16), jnp.float32),
    )(posg, posg, off_p, dst_p.reshape(EP, 1))


def _trec_body(er1_ref, er2_ref, out_ref):
    er1 = er1_ref[...]
    er2 = er2_ref[...]
    v1 = er1[:, 0:3]
    v2 = er2[:, 0:3]
    d1 = jnp.maximum(er1[:, 3:4], 1e-8)
    d2 = jnp.maximum(er2[:, 3:4], 1e-8)
    cos = jnp.clip(jnp.sum(v1 * v2, axis=1, keepdims=True) / (d1 * d2),
                   -1.0, 1.0)
    dd = cos - _centers_a()
    ang = jnp.exp(-(dd * dd) * (1.0 / 0.0225))
    cw2 = er1[:, 4:5] * er2[:, 4:5] * _row_mask(Tt, BT)
    caf = er1[:, 5:6]
    out_ref[...] = jnp.concatenate(
        [ang, d1, d2, cw2, caf, jnp.zeros((BT, 4), jnp.float32)], axis=1)


def _triplet_record(er12):
    grid = TP // BT
    return pl.pallas_call(
        _trec_body,
        grid=(grid,),
        in_specs=[pl.BlockSpec((BT, 16), lambda i: (i, 0)),
                  pl.BlockSpec((BT, 16), lambda i: (i + TP // BT, 0))],
        out_specs=pl.BlockSpec((BT, 24), lambda i: (i, 0)),
        out_shape=jax.ShapeDtypeStruct((TP, 24), jnp.float32),
    )(er12, er12)


def _two_t(gd, gs, erec, W_ref, b_ref):
    dist = erec[:, 3:4]
    ef = _edge_basis(dist)
    t = jnp.dot(gd, W_ref[0:F, :], preferred_element_type=jnp.float32)
    t += jnp.dot(gs, W_ref[F:2 * F, :], preferred_element_type=jnp.float32)
    t += jnp.dot(ef, W_ref[2 * F:, :], preferred_element_type=jnp.float32)
    return t + b_ref[...]


def _two_pass1_body(gd_ref, gs_ref, er_ref, W_ref, b_ref, out_ref):
    i = pl.program_id(0)
    t = _two_t(gd_ref[...], gs_ref[...], er_ref[...], W_ref, b_ref)
    m = _row_mask(Ee, BE)
    t = t * m

    @pl.when(i == 0)
    def _():
        out_ref[...] = jnp.zeros_like(out_ref)

    out_ref[0:1, :] += jnp.sum(t, axis=0, keepdims=True)
    out_ref[1:2, :] += jnp.sum(t * t * m, axis=0, keepdims=True)


def _two_pass2_body(gd_ref, gs_ref, er_ref, W_ref, b_ref, st_ref, g1_ref,
                    be1_ref, out_ref):
    t = _two_t(gd_ref[...], gs_ref[...], er_ref[...], W_ref, b_ref)
    mu = st_ref[0:1, :] * (1.0 / Ee)
    var = st_ref[1:2, :] * (1.0 / Ee) - mu * mu
    tn = (t - mu) / jnp.sqrt(var + 1e-5) * g1_ref[...] + be1_ref[...]
    gate = jax.nn.sigmoid(tn[:, :F])
    core = _softplus(tn[:, F:])
    cw = er_ref[...][:, 4:5]
    out_ref[...] = gate * core * cw


def _two_stage(g2b, erec, W, b, g1, be1):
    grid = EP // BE
    rowd = lambda i: (i, 0)
    rows = lambda i: (i + EP // BE, 0)
    whole = lambda i: (0, 0)
    b2 = b.reshape(1, 2 * F)
    g1r = g1.reshape(1, 2 * F)
    be1r = be1.reshape(1, 2 * F)
    stats = pl.pallas_call(
        _two_pass1_body,
        grid=(grid,),
        in_specs=[pl.BlockSpec((BE, F), rowd), pl.BlockSpec((BE, F), rows),
                  pl.BlockSpec((BE, 16), rowd),
                  pl.BlockSpec((2 * F + G, 2 * F), whole),
                  pl.BlockSpec((1, 2 * F), whole)],
        out_specs=pl.BlockSpec((8, 2 * F), whole),
        out_shape=jax.ShapeDtypeStruct((8, 2 * F), jnp.float32),
    )(g2b, g2b, erec, W, b2)
    msg = pl.pallas_call(
        _two_pass2_body,
        grid=(grid,),
        in_specs=[pl.BlockSpec((BE, F), rowd), pl.BlockSpec((BE, F), rows),
                  pl.BlockSpec((BE, 16), rowd),
                  pl.BlockSpec((2 * F + G, 2 * F), whole),
                  pl.BlockSpec((1, 2 * F), whole),
                  pl.BlockSpec((8, 2 * F), whole),
                  pl.BlockSpec((1, 2 * F), whole),
                  pl.BlockSpec((1, 2 * F), whole)],
        out_specs=pl.BlockSpec((BE, F), rowd),
        out_shape=jax.ShapeDtypeStruct((EP, F), jnp.float32),
    )(g2b, g2b, erec, W, b2, stats, g1r, be1r)
    return msg


def _three_t(gca, trec, Q_ref, b_ref):
    d1 = trec[:, A:A + 1]
    d2 = trec[:, A + 1:A + 2]
    ang = trec[:, 0:A]
    ef1 = _edge_basis(d1)
    ef2 = _edge_basis(d2)
    t = jnp.dot(gca, Q_ref[0:F, :], preferred_element_type=jnp.float32)
    t += jnp.dot(ef1, Q_ref[F:F + G, :], preferred_element_type=jnp.float32)
    t += jnp.dot(ef2, Q_ref[F + G:F + 2 * G, :],
                 preferred_element_type=jnp.float32)
    t += jnp.dot(ang, Q_ref[F + 2 * G:, :], preferred_element_type=jnp.float32)
    return t + b_ref[...]


def _three_pass1_body(gca_ref, tr_ref, Q_ref, b_ref, out_ref):
    i = pl.program_id(0)
    t = _three_t(gca_ref[...], tr_ref[...], Q_ref, b_ref)
    m = _row_mask(Tt, BT)
    t = t * m

    @pl.when(i == 0)
    def _():
        out_ref[...] = jnp.zeros_like(out_ref)

    out_ref[0:1, :] += jnp.sum(t, axis=0, keepdims=True)
    out_ref[1:2, :] += jnp.sum(t * t * m, axis=0, keepdims=True)


def _three_pass2_body(gca_ref, tr_ref, Q_ref, b_ref, st_ref, g1_ref, be1_ref,
                      out_ref):
    t = _three_t(gca_ref[...], tr_ref[...], Q_ref, b_ref)
    mu = st_ref[0:1, :] * (1.0 / Tt)
    var = st_ref[1:2, :] * (1.0 / Tt) - mu * mu
    tn = (t - mu) / jnp.sqrt(var + 1e-5) * g1_ref[...] + be1_ref[...]
    gate = jax.nn.sigmoid(tn[:, :F])
    core = _softplus(tn[:, F:])
    cw2 = tr_ref[...][:, A + 2:A + 3]
    out_ref[...] = gate * core * cw2


def _three_stage(gca, trec, Q, b, g1, be1):
    grid = TP // BT
    row = lambda i: (i, 0)
    whole = lambda i: (0, 0)
    b2 = b.reshape(1, 2 * F)
    g1r = g1.reshape(1, 2 * F)
    be1r = be1.reshape(1, 2 * F)
    K = F + 2 * G + A
    stats = pl.pallas_call(
        _three_pass1_body,
        grid=(grid,),
        in_specs=[pl.BlockSpec((BT, F), row), pl.BlockSpec((BT, 24), row),
                  pl.BlockSpec((K, 2 * F), whole),
                  pl.BlockSpec((1, 2 * F), whole)],
        out_specs=pl.BlockSpec((8, 2 * F), whole),
        out_shape=jax.ShapeDtypeStruct((8, 2 * F), jnp.float32),
    )(gca, trec, Q, b2)
    msg = pl.pallas_call(
        _three_pass2_body,
        grid=(grid,),
        in_specs=[pl.BlockSpec((BT, F), row), pl.BlockSpec((BT, 24), row),
                  pl.BlockSpec((K, 2 * F), whole),
                  pl.BlockSpec((1, 2 * F), whole),
                  pl.BlockSpec((8, 2 * F), whole),
                  pl.BlockSpec((1, 2 * F), whole),
                  pl.BlockSpec((1, 2 * F), whole)],
        out_specs=pl.BlockSpec((BT, F), row),
        out_shape=jax.ShapeDtypeStruct((TP, F), jnp.float32),
    )(gca, trec, Q, b2, stats, g1r, be1r)
    return msg


def _node_body(af_ref, ag_ref, g2_ref, be2_ref, out_ref):
    ag = ag_ref[0:Nn, :] + ag_ref[Nn:2 * Nn, :]
    mu = jnp.mean(ag, axis=0, keepdims=True)
    var = jnp.mean(ag * ag, axis=0, keepdims=True) - mu * mu
    an = (ag - mu) / jnp.sqrt(var + 1e-5) * g2_ref[...] + be2_ref[...]
    out_ref[...] = _softplus(af_ref[...] + an)


def _node_update(af, parts, g2, be2):
    return pl.pallas_call(
        _node_body,
        in_specs=[pl.BlockSpec((Nn, F), lambda: (0, 0)),
                  pl.BlockSpec((2 * Nn, F), lambda: (0, 0)),
                  pl.BlockSpec((1, F), lambda: (0, 0)),
                  pl.BlockSpec((1, F), lambda: (0, 0))],
        out_specs=pl.BlockSpec((Nn, F), lambda: (0, 0)),
        out_shape=jax.ShapeDtypeStruct((Nn, F), jnp.float32),
    )(af, parts.reshape(2 * Nn, F), g2.reshape(1, F), be2.reshape(1, F))


def _embed_body(at_ref, emb_ref, out_ref):
    oh = (at_ref[...] == lax.broadcasted_iota(jnp.int32, (1, 128), 1))
    out_ref[...] = jnp.dot(oh.astype(jnp.float32), emb_ref[...],
                           preferred_element_type=jnp.float32)


def _embed(atom_types, emb):
    embp = jnp.concatenate([emb, jnp.zeros((128 - emb.shape[0], F))], axis=0)
    return pl.pallas_call(
        _embed_body,
        in_specs=[pl.BlockSpec((Nn, 1), lambda: (0, 0)),
                  pl.BlockSpec((128, F), lambda: (0, 0))],
        out_specs=pl.BlockSpec((Nn, F), lambda: (0, 0)),
        out_shape=jax.ShapeDtypeStruct((Nn, F), jnp.float32),
    )(atom_types.reshape(Nn, 1).astype(jnp.int32), embp)


def _head_body(af_ref, batch_ref, W1_ref, b1_ref, W2_ref, b2_ref, W3_ref,
               b3_ref, out_ref):
    h = _softplus(jnp.dot(af_ref[...], W1_ref[...],
                          preferred_element_type=jnp.float32) + b1_ref[...])
    h = _softplus(jnp.dot(h, W2_ref[...],
                          preferred_element_type=jnp.float32) + b2_ref[...])
    e = jnp.dot(h, W3_ref[...], preferred_element_type=jnp.float32) \
        + b3_ref[...]
    mask = (batch_ref[...] == lax.broadcasted_iota(jnp.int32, (1, Bb), 1))
    out_ref[...] = jnp.sum(e * mask.astype(jnp.float32), axis=0,
                           keepdims=True)


def _head(af, batch, params_head):
    (W1, b1), (W2, b2), (W3, b3) = params_head
    whole = lambda: (0, 0)
    out = pl.pallas_call(
        _head_body,
        in_specs=[pl.BlockSpec((Nn, F), whole),
                  pl.BlockSpec((Nn, 1), whole),
                  pl.BlockSpec((F, 128), whole),
                  pl.BlockSpec((1, 128), whole),
                  pl.BlockSpec((128, F), whole),
                  pl.BlockSpec((1, F), whole),
                  pl.BlockSpec((F, 1), whole),
                  pl.BlockSpec((1, 1), whole)],
        out_specs=pl.BlockSpec((1, Bb), whole),
        out_shape=jax.ShapeDtypeStruct((1, Bb), jnp.float32),
    )(af, batch.reshape(Nn, 1), W1, b1.reshape(1, 128), W2,
      b2.reshape(1, F), W3, b3.reshape(1, 1))
    return out.reshape(Bb)


def kernel(atom_types, pos, edge_index, edge_offset, triplet_idx, batch,
           num_atoms, volume, params):
    src = edge_index[0].astype(jnp.int32)
    dst = edge_index[1].astype(jnp.int32)
    e1 = triplet_idx[0].astype(jnp.int32)
    e2 = triplet_idx[1].astype(jnp.int32)

    padE = jnp.zeros((EP - Ee,), jnp.int32)
    src_p = jnp.concatenate([src, padE])
    dst_p = jnp.concatenate([dst, padE])
    e1_p = jnp.concatenate([e1, padE])
    e2_p = jnp.concatenate([e2, padE])

    idx_sd = jnp.concatenate([src_p, dst_p]).reshape(NW, 2 * EP // (NW * CH),
                                                     CH)
    idx_e12 = jnp.concatenate([e1_p, e2_p]).reshape(NW, 2 * TP // (NW * CH),
                                                    CH)
    idx_ds = jnp.concatenate([dst_p, src_p]).reshape(NW, 2 * EP // (NW * CH),
                                                     CH)
    idx_dst = dst_p.reshape(NW, EP // (NW * CH), CH)
    off_p = jnp.concatenate([edge_offset,
                             jnp.zeros((EP - Ee, 3), jnp.float32)])
    pos16 = jnp.concatenate([pos, jnp.zeros((Nn, 13), jnp.float32)], axis=1)
    zero_n = jnp.zeros((Nn, F), jnp.float32)

    posg = _sc_gather(pos16, idx_sd, 16, 8)
    erec = _edge_record(posg, off_p, dst_p)
    er12 = _sc_gather(erec, idx_e12, 16, 8)
    trec = _triplet_record(er12)
    ca_p = trec[:, A + 3].astype(jnp.int32)
    idx_ca = ca_p.reshape(NW, TP // (NW * CH), CH)

    af = _embed(atom_types, params['embed'])
    for l in range(NC):
        p = params['two'][l]
        g2b = _sc_gather(af, idx_ds, F, 4)
        msg = _two_stage(g2b, erec, p['W'], p['b'], p['g1'], p['be1'])
        parts = _sc_scatter_add(msg, idx_dst, zero_n)
        af = _node_update(af, parts, p['g2'], p['be2'])

        q = params['three'][l]
        gca = _sc_gather(af, idx_ca, F, 4)
        msg2 = _three_stage(gca, trec, q['W'], q['b'], q['g1'], q['be1'])
        parts2 = _sc_scatter_add(msg2, idx_ca, zero_n)
        af = _node_update(af, parts2, q['g2'], q['be2'])

    return _head(af, batch, params['head'])


# speedup vs baseline: 2.1902x; 1.2188x over previous
import functools
import math

import jax
import jax.numpy as jnp
from jax import lax
from jax.experimental import pallas as pl
from jax.experimental.pallas import tpu as pltpu
from jax.experimental.pallas import tpu_sc as plsc

Nn = 10000
Ee = 320000
Tt = 320000
Bb = 4
F = 64
G = 80
A = 16
NC = 2
CUT = 5.0

EP = 327680
TP = 327680
BE = 2048
BT = 2048
NW = 32
CH = 128


@functools.lru_cache(None)
def _make_sc_gather(n_dma, D, k):
    M = NW * n_dma * CH
    mesh = plsc.VectorSubcoreMesh(core_axis_name="c", subcore_axis_name="s")

    def body(table_hbm, idx_hbm, out_hbm, idx_v, bufs, semg, semo):
        wid = lax.axis_index("s") * 2 + lax.axis_index("c")
        pltpu.sync_copy(idx_hbm.at[wid], idx_v)
        base = wid * (n_dma * CH)
        out_h = [None, None]
        g = 0
        j0 = 0
        while j0 < n_dma:
            par = g % 2
            kk = min(k, n_dma - j0)
            if out_h[par] is not None:
                out_h[par].wait()
            hs = []
            for i in range(kk):
                hs.append(pltpu.async_copy(
                    table_hbm.at[idx_v.at[j0 + i]],
                    bufs.at[par, pl.ds(i * CH, CH)], semg))
            for h in hs:
                h.wait()
            out_h[par] = pltpu.async_copy(
                bufs.at[par, pl.ds(0, kk * CH)],
                out_hbm.at[pl.ds(base + j0 * CH, kk * CH)], semo)
            g += 1
            j0 += kk
        for h in out_h:
            if h is not None:
                h.wait()

    return pl.kernel(
        body,
        out_type=jax.ShapeDtypeStruct((M, D), jnp.float32),
        mesh=mesh,
        compiler_params=pltpu.CompilerParams(use_tc_tiling_on_sc=False),
        scratch_types=[pltpu.VMEM((n_dma, CH), jnp.int32),
                       pltpu.VMEM((2, k * CH, D), jnp.float32),
                       pltpu.SemaphoreType.DMA,
                       pltpu.SemaphoreType.DMA])


@functools.lru_cache(None)
def _make_sc_gather_spmem(n_dma, D, k, nrow):
    M = NW * n_dma * CH
    mesh = plsc.VectorSubcoreMesh(core_axis_name="c", subcore_axis_name="s")
    rps = nrow // 16

    def body(table_hbm, idx_hbm, out_hbm, idx_v, tbl, bufs, semg, semo):
        s = lax.axis_index("s")
        wid = s * 2 + lax.axis_index("c")
        pltpu.sync_copy(idx_hbm.at[wid], idx_v)
        pltpu.sync_copy(table_hbm.at[pl.ds(s * rps, rps)],
                        tbl.at[pl.ds(s * rps, rps)])
        plsc.subcore_barrier()
        base = wid * (n_dma * CH)
        out_h = [None, None]
        g = 0
        j0 = 0
        while j0 < n_dma:
            par = g % 2
            kk = min(k, n_dma - j0)
            if out_h[par] is not None:
                out_h[par].wait()
            hs = []
            for i in range(kk):
                hs.append(pltpu.async_copy(
                    tbl.at[idx_v.at[j0 + i]],
                    bufs.at[par, pl.ds(i * CH, CH)], semg))
            for h in hs:
                h.wait()
            out_h[par] = pltpu.async_copy(
                bufs.at[par, pl.ds(0, kk * CH)],
                out_hbm.at[pl.ds(base + j0 * CH, kk * CH)], semo)
            g += 1
            j0 += kk
        for h in out_h:
            if h is not None:
                h.wait()

    return pl.kernel(
        body,
        out_type=jax.ShapeDtypeStruct((M, D), jnp.float32),
        mesh=mesh,
        compiler_params=pltpu.CompilerParams(use_tc_tiling_on_sc=False),
        scratch_types=[pltpu.VMEM((n_dma, CH), jnp.int32),
                       pltpu.VMEM_SHARED((nrow, D), jnp.float32),
                       pltpu.VMEM((2, k * CH, D), jnp.float32),
                       pltpu.SemaphoreType.DMA,
                       pltpu.SemaphoreType.DMA])


@functools.lru_cache(None)
def _make_sc_scatter_add(n_dma, Nrow, D):
    mesh = plsc.VectorSubcoreMesh(core_axis_name="c", subcore_axis_name="s")
    rps = Nrow // 16

    def body(msg_hbm, idx_hbm, zero_hbm, out_hbm, idx_v, bufs, acc, sem):
        c = lax.axis_index("c")
        s = lax.axis_index("s")
        wid = s * 2 + c
        pltpu.sync_copy(idx_hbm.at[wid], idx_v)
        pltpu.sync_copy(zero_hbm.at[pl.ds(s * rps, rps)],
                        acc.at[pl.ds(s * rps, rps)])
        plsc.subcore_barrier()
        base = wid * (n_dma * CH)
        h = [None, None]
        h[0] = pltpu.async_copy(msg_hbm.at[pl.ds(base, CH)], bufs.at[0], sem)
        for j in range(n_dma):
            par = j % 2
            if j + 1 < n_dma:
                h[1 - par] = pltpu.async_copy(
                    msg_hbm.at[pl.ds(base + (j + 1) * CH, CH)],
                    bufs.at[1 - par], sem)
            h[par].wait()
            pltpu.sync_copy(bufs.at[par], acc.at[idx_v.at[j]], add=True)
        plsc.subcore_barrier()
        pltpu.sync_copy(acc.at[pl.ds(s * rps, rps)],
                        out_hbm.at[c, pl.ds(s * rps, rps)])

    return pl.kernel(
        body,
        out_type=jax.ShapeDtypeStruct((2, Nrow, D), jnp.float32),
        mesh=mesh,
        compiler_params=pltpu.CompilerParams(use_tc_tiling_on_sc=False),
        scratch_types=[pltpu.VMEM((n_dma, CH), jnp.int32),
                       pltpu.VMEM((2, CH, D), jnp.float32),
                       pltpu.VMEM_SHARED((Nrow, D), jnp.float32),
                       pltpu.SemaphoreType.DMA])


def _sc_gather(table, idx3, D, k):
    return _make_sc_gather(idx3.shape[1], D, k)(table, idx3)


def _sc_gather_small(table, idx3, D, k):
    return _make_sc_gather_spmem(idx3.shape[1], D, k,
                                 table.shape[0])(table, idx3)


def _sc_scatter_add(msg, idx3, zero):
    return _make_sc_scatter_add(idx3.shape[1], Nn, F)(msg, idx3, zero)


def _centers_r():
    i = lax.broadcasted_iota(jnp.int32, (1, G), 1).astype(jnp.float32)
    return i * (CUT / (G - 1))


def _centers_a():
    i = lax.broadcasted_iota(jnp.int32, (1, A), 1).astype(jnp.float32)
    return i * (2.0 / (A - 1)) - 1.0


def _softplus(x):
    return jnp.logaddexp(x, 0.0)


def _edge_basis(dist_col):
    d = dist_col - _centers_r()
    return jnp.exp(-(d * d) * 25.0)


def _row_mask(limit, nrows):
    i = pl.program_id(0)
    rid = lax.broadcasted_iota(jnp.int32, (nrows, 1), 0) + i * nrows
    return (rid < limit).astype(jnp.float32)


def _erec_body(pos_s_ref, pos_d_ref, off_ref, dst_ref, out_ref):
    r = pos_s_ref[...][:, 0:3] - pos_d_ref[...][:, 0:3] + off_ref[...]
    d2 = jnp.sum(r * r, axis=1, keepdims=True)
    dist = jnp.sqrt(d2 + 1e-12)
    cw = 0.5 * (jnp.cos(dist * (math.pi / CUT)) + 1.0)
    cw = cw * _row_mask(Ee, BE)
    dstf = dst_ref[...].astype(jnp.float32)
    out_ref[...] = jnp.concatenate(
        [r, dist, cw, dstf, jnp.zeros((BE, 10), jnp.float32)], axis=1)


def _edge_record(posg, off_p, dst_p):
    grid = EP // BE
    return pl.pallas_call(
        _erec_body,
        grid=(grid,),
        in_specs=[pl.BlockSpec((BE, 16), lambda i: (i, 0)),
                  pl.BlockSpec((BE, 16), lambda i: (i + EP // BE, 0)),
                  pl.BlockSpec((BE, 3), lambda i: (i, 0)),
                  pl.BlockSpec((BE, 1), lambda i: (i, 0))],
        out_specs=pl.BlockSpec((BE, 16), lambda i: (i, 0)),
        out_shape=jax.ShapeDtypeStruct((EP, 16), jnp.float32),
    )(posg, posg, off_p, dst_p.reshape(EP, 1))


def _trec_body(er1_ref, er2_ref, out_ref):
    er1 = er1_ref[...]
    er2 = er2_ref[...]
    v1 = er1[:, 0:3]
    v2 = er2[:, 0:3]
    d1 = jnp.maximum(er1[:, 3:4], 1e-8)
    d2 = jnp.maximum(er2[:, 3:4], 1e-8)
    cos = jnp.clip(jnp.sum(v1 * v2, axis=1, keepdims=True) / (d1 * d2),
                   -1.0, 1.0)
    dd = cos - _centers_a()
    ang = jnp.exp(-(dd * dd) * (1.0 / 0.0225))
    cw2 = er1[:, 4:5] * er2[:, 4:5] * _row_mask(Tt, BT)
    caf = er1[:, 5:6]
    out_ref[...] = jnp.concatenate(
        [ang, d1, d2, cw2, caf, jnp.zeros((BT, 4), jnp.float32)], axis=1)


def _triplet_record(er12):
    grid = TP // BT
    return pl.pallas_call(
        _trec_body,
        grid=(grid,),
        in_specs=[pl.BlockSpec((BT, 16), lambda i: (i, 0)),
                  pl.BlockSpec((BT, 16), lambda i: (i + TP // BT, 0))],
        out_specs=pl.BlockSpec((BT, 24), lambda i: (i, 0)),
        out_shape=jax.ShapeDtypeStruct((TP, 24), jnp.float32),
    )(er12, er12)


def _two_t(gd, gs, erec, W_ref, b_ref):
    dist = erec[:, 3:4]
    ef = _edge_basis(dist)
    t = jnp.dot(gd, W_ref[0:F, :], preferred_element_type=jnp.float32)
    t += jnp.dot(gs, W_ref[F:2 * F, :], preferred_element_type=jnp.float32)
    t += jnp.dot(ef, W_ref[2 * F:, :], preferred_element_type=jnp.float32)
    return t + b_ref[...]


def _two_pass1_body(gd_ref, gs_ref, er_ref, W_ref, b_ref, st_ref, t_ref):
    i = pl.program_id(0)
    t = _two_t(gd_ref[...], gs_ref[...], er_ref[...], W_ref, b_ref)
    t_ref[...] = t
    m = _row_mask(Ee, BE)
    t = t * m

    @pl.when(i == 0)
    def _():
        st_ref[...] = jnp.zeros_like(st_ref)

    st_ref[0:1, :] += jnp.sum(t, axis=0, keepdims=True)
    st_ref[1:2, :] += jnp.sum(t * t * m, axis=0, keepdims=True)


def _pass2_body(nrm, cw_col, t_ref, er_ref, st_ref, g1_ref, be1_ref, out_ref):
    t = t_ref[...]
    mu = st_ref[0:1, :] * nrm
    var = st_ref[1:2, :] * nrm - mu * mu
    tn = (t - mu) / jnp.sqrt(var + 1e-5) * g1_ref[...] + be1_ref[...]
    gate = jax.nn.sigmoid(tn[:, :F])
    core = _softplus(tn[:, F:])
    cw = er_ref[...][:, cw_col:cw_col + 1]
    out_ref[...] = gate * core * cw


def _pass2(t_all, er_all, ew, stats, g1, be1, nrm, cw_col):
    grid = EP // BE
    row = lambda i: (i, 0)
    whole = lambda i: (0, 0)
    return pl.pallas_call(
        functools.partial(_pass2_body, nrm, cw_col),
        grid=(grid,),
        in_specs=[pl.BlockSpec((BE, 2 * F), row),
                  pl.BlockSpec((BE, ew), row),
                  pl.BlockSpec((8, 2 * F), whole),
                  pl.BlockSpec((1, 2 * F), whole),
                  pl.BlockSpec((1, 2 * F), whole)],
        out_specs=pl.BlockSpec((BE, F), row),
        out_shape=jax.ShapeDtypeStruct((EP, F), jnp.float32),
    )(t_all, er_all, stats, g1.reshape(1, 2 * F), be1.reshape(1, 2 * F))


def _two_stage(g2b, erec, W, b, g1, be1):
    grid = EP // BE
    rowd = lambda i: (i, 0)
    rows = lambda i: (i + EP // BE, 0)
    whole = lambda i: (0, 0)
    b2 = b.reshape(1, 2 * F)
    stats, t_all = pl.pallas_call(
        _two_pass1_body,
        grid=(grid,),
        in_specs=[pl.BlockSpec((BE, F), rowd), pl.BlockSpec((BE, F), rows),
                  pl.BlockSpec((BE, 16), rowd),
                  pl.BlockSpec((2 * F + G, 2 * F), whole),
                  pl.BlockSpec((1, 2 * F), whole)],
        out_specs=[pl.BlockSpec((8, 2 * F), whole),
                   pl.BlockSpec((BE, 2 * F), rowd)],
        out_shape=[jax.ShapeDtypeStruct((8, 2 * F), jnp.float32),
                   jax.ShapeDtypeStruct((EP, 2 * F), jnp.float32)],
    )(g2b, g2b, erec, W, b2)
    return _pass2(t_all, erec, 16, stats, g1, be1, 1.0 / Ee, 4)


def _three_t(gca, trec, Q_ref, b_ref):
    d1 = trec[:, A:A + 1]
    d2 = trec[:, A + 1:A + 2]
    ang = trec[:, 0:A]
    ef1 = _edge_basis(d1)
    ef2 = _edge_basis(d2)
    t = jnp.dot(gca, Q_ref[0:F, :], preferred_element_type=jnp.float32)
    t += jnp.dot(ef1, Q_ref[F:F + G, :], preferred_element_type=jnp.float32)
    t += jnp.dot(ef2, Q_ref[F + G:F + 2 * G, :],
                 preferred_element_type=jnp.float32)
    t += jnp.dot(ang, Q_ref[F + 2 * G:, :], preferred_element_type=jnp.float32)
    return t + b_ref[...]


def _three_pass1_body(gca_ref, tr_ref, Q_ref, b_ref, st_ref, t_ref):
    i = pl.program_id(0)
    t = _three_t(gca_ref[...], tr_ref[...], Q_ref, b_ref)
    t_ref[...] = t
    m = _row_mask(Tt, BT)
    t = t * m

    @pl.when(i == 0)
    def _():
        st_ref[...] = jnp.zeros_like(st_ref)

    st_ref[0:1, :] += jnp.sum(t, axis=0, keepdims=True)
    st_ref[1:2, :] += jnp.sum(t * t * m, axis=0, keepdims=True)


def _three_stage(gca, trec, Q, b, g1, be1):
    grid = TP // BT
    row = lambda i: (i, 0)
    whole = lambda i: (0, 0)
    b2 = b.reshape(1, 2 * F)
    K = F + 2 * G + A
    stats, t_all = pl.pallas_call(
        _three_pass1_body,
        grid=(grid,),
        in_specs=[pl.BlockSpec((BT, F), row), pl.BlockSpec((BT, 24), row),
                  pl.BlockSpec((K, 2 * F), whole),
                  pl.BlockSpec((1, 2 * F), whole)],
        out_specs=[pl.BlockSpec((8, 2 * F), whole),
                   pl.BlockSpec((BT, 2 * F), row)],
        out_shape=[jax.ShapeDtypeStruct((8, 2 * F), jnp.float32),
                   jax.ShapeDtypeStruct((TP, 2 * F), jnp.float32)],
    )(gca, trec, Q, b2)
    return _pass2(t_all, trec, 24, stats, g1, be1, 1.0 / Tt, A + 2)


def _node_body(af_ref, ag_ref, g2_ref, be2_ref, out_ref):
    ag = ag_ref[0:Nn, :] + ag_ref[Nn:2 * Nn, :]
    mu = jnp.mean(ag, axis=0, keepdims=True)
    var = jnp.mean(ag * ag, axis=0, keepdims=True) - mu * mu
    an = (ag - mu) / jnp.sqrt(var + 1e-5) * g2_ref[...] + be2_ref[...]
    out_ref[...] = _softplus(af_ref[...] + an)


def _node_update(af, parts, g2, be2):
    return pl.pallas_call(
        _node_body,
        in_specs=[pl.BlockSpec((Nn, F), lambda: (0, 0)),
                  pl.BlockSpec((2 * Nn, F), lambda: (0, 0)),
                  pl.BlockSpec((1, F), lambda: (0, 0)),
                  pl.BlockSpec((1, F), lambda: (0, 0))],
        out_specs=pl.BlockSpec((Nn, F), lambda: (0, 0)),
        out_shape=jax.ShapeDtypeStruct((Nn, F), jnp.float32),
    )(af, parts.reshape(2 * Nn, F), g2.reshape(1, F), be2.reshape(1, F))


def _embed_body(at_ref, emb_ref, out_ref):
    oh = (at_ref[...] == lax.broadcasted_iota(jnp.int32, (1, 128), 1))
    out_ref[...] = jnp.dot(oh.astype(jnp.float32), emb_ref[...],
                           preferred_element_type=jnp.float32)


def _embed(atom_types, emb):
    embp = jnp.concatenate([emb, jnp.zeros((128 - emb.shape[0], F))], axis=0)
    return pl.pallas_call(
        _embed_body,
        in_specs=[pl.BlockSpec((Nn, 1), lambda: (0, 0)),
                  pl.BlockSpec((128, F), lambda: (0, 0))],
        out_specs=pl.BlockSpec((Nn, F), lambda: (0, 0)),
        out_shape=jax.ShapeDtypeStruct((Nn, F), jnp.float32),
    )(atom_types.reshape(Nn, 1).astype(jnp.int32), embp)


def _head_body(af_ref, batch_ref, W1_ref, b1_ref, W2_ref, b2_ref, W3_ref,
               b3_ref, out_ref):
    h = _softplus(jnp.dot(af_ref[...], W1_ref[...],
                          preferred_element_type=jnp.float32) + b1_ref[...])
    h = _softplus(jnp.dot(h, W2_ref[...],
                          preferred_element_type=jnp.float32) + b2_ref[...])
    e = jnp.dot(h, W3_ref[...], preferred_element_type=jnp.float32) \
        + b3_ref[...]
    mask = (batch_ref[...] == lax.broadcasted_iota(jnp.int32, (1, Bb), 1))
    out_ref[...] = jnp.sum(e * mask.astype(jnp.float32), axis=0,
                           keepdims=True)


def _head(af, batch, params_head):
    (W1, b1), (W2, b2), (W3, b3) = params_head
    whole = lambda: (0, 0)
    out = pl.pallas_call(
        _head_body,
        in_specs=[pl.BlockSpec((Nn, F), whole),
                  pl.BlockSpec((Nn, 1), whole),
                  pl.BlockSpec((F, 128), whole),
                  pl.BlockSpec((1, 128), whole),
                  pl.BlockSpec((128, F), whole),
                  pl.BlockSpec((1, F), whole),
                  pl.BlockSpec((F, 1), whole),
                  pl.BlockSpec((1, 1), whole)],
        out_specs=pl.BlockSpec((1, Bb), whole),
        out_shape=jax.ShapeDtypeStruct((1, Bb), jnp.float32),
    )(af, batch.reshape(Nn, 1), W1, b1.reshape(1, 128), W2,
      b2.reshape(1, F), W3, b3.reshape(1, 1))
    return out.reshape(Bb)


def kernel(atom_types, pos, edge_index, edge_offset, triplet_idx, batch,
           num_atoms, volume, params):
    src = edge_index[0].astype(jnp.int32)
    dst = edge_index[1].astype(jnp.int32)
    e1 = triplet_idx[0].astype(jnp.int32)
    e2 = triplet_idx[1].astype(jnp.int32)

    padE = jnp.zeros((EP - Ee,), jnp.int32)
    src_p = jnp.concatenate([src, padE])
    dst_p = jnp.concatenate([dst, padE])
    e1_p = jnp.concatenate([e1, padE])
    e2_p = jnp.concatenate([e2, padE])

    idx_sd = jnp.concatenate([src_p, dst_p]).reshape(NW, 2 * EP // (NW * CH),
                                                     CH)
    idx_e12 = jnp.concatenate([e1_p, e2_p]).reshape(NW, 2 * TP // (NW * CH),
                                                    CH)
    idx_ds = jnp.concatenate([dst_p, src_p]).reshape(NW, 2 * EP // (NW * CH),
                                                     CH)
    idx_dst = dst_p.reshape(NW, EP // (NW * CH), CH)
    off_p = jnp.concatenate([edge_offset,
                             jnp.zeros((EP - Ee, 3), jnp.float32)])
    pos16 = jnp.concatenate([pos, jnp.zeros((Nn, 13), jnp.float32)], axis=1)
    zero_n = jnp.zeros((Nn, F), jnp.float32)

    posg = _sc_gather_small(pos16, idx_sd, 16, 8)
    erec = _edge_record(posg, off_p, dst_p)
    er12 = _sc_gather(erec, idx_e12, 16, 16)
    trec = _triplet_record(er12)
    ca_p = trec[:, A + 3].astype(jnp.int32)
    idx_ca = ca_p.reshape(NW, TP // (NW * CH), CH)

    af = _embed(atom_types, params['embed'])
    for l in range(NC):
        p = params['two'][l]
        g2b = _sc_gather_small(af, idx_ds, F, 4)
        msg = _two_stage(g2b, erec, p['W'], p['b'], p['g1'], p['be1'])
        parts = _sc_scatter_add(msg, idx_dst, zero_n)
        af = _node_update(af, parts, p['g2'], p['be2'])

        q = params['three'][l]
        gca = _sc_gather_small(af, idx_ca, F, 4)
        msg2 = _three_stage(gca, trec, q['W'], q['b'], q['g1'], q['be1'])
        parts2 = _sc_scatter_add(msg2, idx_ca, zero_n)
        af = _node_update(af, parts2, q['g2'], q['be2'])

    return _head(af, batch, params['head'])


# speedup vs baseline: 2.1907x; 1.0002x over previous
import functools
import math

import jax
import jax.numpy as jnp
from jax import lax
from jax.experimental import pallas as pl
from jax.experimental.pallas import tpu as pltpu
from jax.experimental.pallas import tpu_sc as plsc

Nn = 10000
Ee = 320000
Tt = 320000
Bb = 4
F = 64
G = 80
A = 16
NC = 2
CUT = 5.0

EP = 327680
TP = 327680
BE = 2048
BT = 2048
NW = 32
CH = 128


@functools.lru_cache(None)
def _make_sc_gather(n_dma, D, k):
    M = NW * n_dma * CH
    mesh = plsc.VectorSubcoreMesh(core_axis_name="c", subcore_axis_name="s")

    def body(table_hbm, idx_hbm, out_hbm, idx_v, bufs, semg, semo):
        wid = lax.axis_index("s") * 2 + lax.axis_index("c")
        pltpu.sync_copy(idx_hbm.at[wid], idx_v)
        base = wid * (n_dma * CH)
        out_h = [None, None]
        g = 0
        j0 = 0
        while j0 < n_dma:
            par = g % 2
            kk = min(k, n_dma - j0)
            if out_h[par] is not None:
                out_h[par].wait()
            hs = []
            for i in range(kk):
                hs.append(pltpu.async_copy(
                    table_hbm.at[idx_v.at[j0 + i]],
                    bufs.at[par, pl.ds(i * CH, CH)], semg))
            for h in hs:
                h.wait()
            out_h[par] = pltpu.async_copy(
                bufs.at[par, pl.ds(0, kk * CH)],
                out_hbm.at[pl.ds(base + j0 * CH, kk * CH)], semo)
            g += 1
            j0 += kk
        for h in out_h:
            if h is not None:
                h.wait()

    return pl.kernel(
        body,
        out_type=jax.ShapeDtypeStruct((M, D), jnp.float32),
        mesh=mesh,
        compiler_params=pltpu.CompilerParams(use_tc_tiling_on_sc=False),
        scratch_types=[pltpu.VMEM((n_dma, CH), jnp.int32),
                       pltpu.VMEM((2, k * CH, D), jnp.float32),
                       pltpu.SemaphoreType.DMA,
                       pltpu.SemaphoreType.DMA])


@functools.lru_cache(None)
def _make_sc_gather_spmem(n_dma, D, k, nrow):
    M = NW * n_dma * CH
    mesh = plsc.VectorSubcoreMesh(core_axis_name="c", subcore_axis_name="s")
    rps = nrow // 16

    def body(table_hbm, idx_hbm, out_hbm, idx_v, tbl, bufs, semg, semo):
        s = lax.axis_index("s")
        wid = s * 2 + lax.axis_index("c")
        pltpu.sync_copy(idx_hbm.at[wid], idx_v)
        pltpu.sync_copy(table_hbm.at[pl.ds(s * rps, rps)],
                        tbl.at[pl.ds(s * rps, rps)])
        plsc.subcore_barrier()
        base = wid * (n_dma * CH)
        out_h = [None, None]
        g = 0
        j0 = 0
        while j0 < n_dma:
            par = g % 2
            kk = min(k, n_dma - j0)
            if out_h[par] is not None:
                out_h[par].wait()
            hs = []
            for i in range(kk):
                hs.append(pltpu.async_copy(
                    tbl.at[idx_v.at[j0 + i]],
                    bufs.at[par, pl.ds(i * CH, CH)], semg))
            for h in hs:
                h.wait()
            out_h[par] = pltpu.async_copy(
                bufs.at[par, pl.ds(0, kk * CH)],
                out_hbm.at[pl.ds(base + j0 * CH, kk * CH)], semo)
            g += 1
            j0 += kk
        for h in out_h:
            if h is not None:
                h.wait()

    return pl.kernel(
        body,
        out_type=jax.ShapeDtypeStruct((M, D), jnp.float32),
        mesh=mesh,
        compiler_params=pltpu.CompilerParams(use_tc_tiling_on_sc=False),
        scratch_types=[pltpu.VMEM((n_dma, CH), jnp.int32),
                       pltpu.VMEM_SHARED((nrow, D), jnp.float32),
                       pltpu.VMEM((2, k * CH, D), jnp.float32),
                       pltpu.SemaphoreType.DMA,
                       pltpu.SemaphoreType.DMA])


@functools.lru_cache(None)
def _make_sc_scatter_add(n_dma, Nrow, D):
    mesh = plsc.VectorSubcoreMesh(core_axis_name="c", subcore_axis_name="s")
    rps = Nrow // 16

    def body(msg_hbm, idx_hbm, zero_hbm, out_hbm, idx_v, bufs, acc, sem):
        c = lax.axis_index("c")
        s = lax.axis_index("s")
        wid = s * 2 + c
        pltpu.sync_copy(idx_hbm.at[wid], idx_v)
        pltpu.sync_copy(zero_hbm.at[pl.ds(s * rps, rps)],
                        acc.at[pl.ds(s * rps, rps)])
        plsc.subcore_barrier()
        base = wid * (n_dma * CH)
        h = [None, None]
        h[0] = pltpu.async_copy(msg_hbm.at[pl.ds(base, CH)], bufs.at[0], sem)
        for j in range(n_dma):
            par = j % 2
            if j + 1 < n_dma:
                h[1 - par] = pltpu.async_copy(
                    msg_hbm.at[pl.ds(base + (j + 1) * CH, CH)],
                    bufs.at[1 - par], sem)
            h[par].wait()
            pltpu.sync_copy(bufs.at[par], acc.at[idx_v.at[j]], add=True)
        plsc.subcore_barrier()
        pltpu.sync_copy(acc.at[pl.ds(s * rps, rps)],
                        out_hbm.at[c, pl.ds(s * rps, rps)])

    return pl.kernel(
        body,
        out_type=jax.ShapeDtypeStruct((2, Nrow, D), jnp.float32),
        mesh=mesh,
        compiler_params=pltpu.CompilerParams(use_tc_tiling_on_sc=False),
        scratch_types=[pltpu.VMEM((n_dma, CH), jnp.int32),
                       pltpu.VMEM((2, CH, D), jnp.float32),
                       pltpu.VMEM_SHARED((Nrow, D), jnp.float32),
                       pltpu.SemaphoreType.DMA])


def _sc_gather(table, idx3, D, k):
    return _make_sc_gather(idx3.shape[1], D, k)(table, idx3)


def _sc_gather_small(table, idx3, D, k):
    return _make_sc_gather_spmem(idx3.shape[1], D, k,
                                 table.shape[0])(table, idx3)


def _sc_scatter_add(msg, idx3, zero):
    return _make_sc_scatter_add(idx3.shape[1], Nn, F)(msg, idx3, zero)


def _centers_r():
    i = lax.broadcasted_iota(jnp.int32, (1, G), 1).astype(jnp.float32)
    return i * (CUT / (G - 1))


def _centers_a():
    i = lax.broadcasted_iota(jnp.int32, (1, A), 1).astype(jnp.float32)
    return i * (2.0 / (A - 1)) - 1.0


def _softplus(x):
    return jnp.logaddexp(x, 0.0)


def _edge_basis(dist_col):
    d = dist_col - _centers_r()
    return jnp.exp(-(d * d) * 25.0)


def _row_mask(limit, nrows):
    i = pl.program_id(0)
    rid = lax.broadcasted_iota(jnp.int32, (nrows, 1), 0) + i * nrows
    return (rid < limit).astype(jnp.float32)


def _erec_body(pos_s_ref, pos_d_ref, off_ref, dst_ref, out_ref):
    r = pos_s_ref[...][:, 0:3] - pos_d_ref[...][:, 0:3] + off_ref[...]
    d2 = jnp.sum(r * r, axis=1, keepdims=True)
    dist = jnp.sqrt(d2 + 1e-12)
    cw = 0.5 * (jnp.cos(dist * (math.pi / CUT)) + 1.0)
    cw = cw * _row_mask(Ee, BE)
    dstf = dst_ref[...].astype(jnp.float32)
    out_ref[...] = jnp.concatenate(
        [r, dist, cw, dstf, jnp.zeros((BE, 10), jnp.float32)], axis=1)


def _edge_record(posg, off_p, dst_p):
    grid = EP // BE
    return pl.pallas_call(
        _erec_body,
        grid=(grid,),
        in_specs=[pl.BlockSpec((BE, 16), lambda i: (i, 0)),
                  pl.BlockSpec((BE, 16), lambda i: (i + EP // BE, 0)),
                  pl.BlockSpec((BE, 3), lambda i: (i, 0)),
                  pl.BlockSpec((BE, 1), lambda i: (i, 0))],
        out_specs=pl.BlockSpec((BE, 16), lambda i: (i, 0)),
        out_shape=jax.ShapeDtypeStruct((EP, 16), jnp.float32),
    )(posg, posg, off_p, dst_p.reshape(EP, 1))


def _trec_body(er1_ref, er2_ref, out_ref):
    er1 = er1_ref[...]
    er2 = er2_ref[...]
    v1 = er1[:, 0:3]
    v2 = er2[:, 0:3]
    d1 = jnp.maximum(er1[:, 3:4], 1e-8)
    d2 = jnp.maximum(er2[:, 3:4], 1e-8)
    cos = jnp.clip(jnp.sum(v1 * v2, axis=1, keepdims=True) / (d1 * d2),
                   -1.0, 1.0)
    dd = cos - _centers_a()
    ang = jnp.exp(-(dd * dd) * (1.0 / 0.0225))
    cw2 = er1[:, 4:5] * er2[:, 4:5] * _row_mask(Tt, BT)
    caf = er1[:, 5:6]
    out_ref[...] = jnp.concatenate(
        [ang, d1, d2, cw2, caf, jnp.zeros((BT, 4), jnp.float32)], axis=1)


def _triplet_record(er12):
    grid = TP // BT
    return pl.pallas_call(
        _trec_body,
        grid=(grid,),
        in_specs=[pl.BlockSpec((BT, 16), lambda i: (i, 0)),
                  pl.BlockSpec((BT, 16), lambda i: (i + TP // BT, 0))],
        out_specs=pl.BlockSpec((BT, 24), lambda i: (i, 0)),
        out_shape=jax.ShapeDtypeStruct((TP, 24), jnp.float32),
    )(er12, er12)


def _two_t(gd, gs, erec, W_ref, b_ref):
    dist = erec[:, 3:4]
    ef = _edge_basis(dist)
    t = jnp.dot(gd, W_ref[0:F, :], preferred_element_type=jnp.float32)
    t += jnp.dot(gs, W_ref[F:2 * F, :], preferred_element_type=jnp.float32)
    t += jnp.dot(ef, W_ref[2 * F:, :], preferred_element_type=jnp.float32)
    return t + b_ref[...]


def _two_pass1_body(gd_ref, gs_ref, er_ref, W_ref, b_ref, st_ref, t_ref):
    i = pl.program_id(0)
    t = _two_t(gd_ref[...], gs_ref[...], er_ref[...], W_ref, b_ref)
    t_ref[...] = t
    rid = lax.broadcasted_iota(jnp.int32, (BE, 2 * F), 0) + i * BE
    tm = jnp.where(rid < Ee, t, 0.0)

    @pl.when(i == 0)
    def _():
        st_ref[...] = jnp.zeros_like(st_ref)

    st_ref[0:1, :] += jnp.sum(tm, axis=0, keepdims=True)
    st_ref[1:2, :] += jnp.sum(tm * tm, axis=0, keepdims=True)


def _pass2_body(nrm, cw_col, t_ref, er_ref, st_ref, g1_ref, be1_ref,
                out_ref):
    t = t_ref[...]
    mu = st_ref[0:1, :] * nrm
    var = st_ref[1:2, :] * nrm - mu * mu
    tn = (t - mu) / jnp.sqrt(var + 1e-5) * g1_ref[...] + be1_ref[...]
    gate = jax.nn.sigmoid(tn[:, :F])
    core = _softplus(tn[:, F:])
    cw = er_ref[...][:, cw_col:cw_col + 1]
    out_ref[...] = gate * core * cw


def _pass2(t_all, er_all, stats, g1, be1, nrm, cw_col):
    grid = EP // BE
    row = lambda i: (i, 0)
    whole = lambda i: (0, 0)
    er_spec = pl.BlockSpec((BE, er_all.shape[1]), row)
    return pl.pallas_call(
        functools.partial(_pass2_body, nrm, cw_col),
        grid=(grid,),
        in_specs=[pl.BlockSpec((BE, 2 * F), row),
                  er_spec,
                  pl.BlockSpec((8, 2 * F), whole),
                  pl.BlockSpec((1, 2 * F), whole),
                  pl.BlockSpec((1, 2 * F), whole)],
        out_specs=pl.BlockSpec((BE, F), row),
        out_shape=jax.ShapeDtypeStruct((EP, F), jnp.float32),
    )(t_all, er_all, stats, g1.reshape(1, 2 * F), be1.reshape(1, 2 * F))


def _two_stage(g2b, erec, W, b, g1, be1):
    grid = EP // BE
    rowd = lambda i: (i, 0)
    rows = lambda i: (i + EP // BE, 0)
    whole = lambda i: (0, 0)
    b2 = b.reshape(1, 2 * F)
    stats, t_all = pl.pallas_call(
        _two_pass1_body,
        grid=(grid,),
        in_specs=[pl.BlockSpec((BE, F), rowd),
                  pl.BlockSpec((BE, F), rows),
                  pl.BlockSpec((BE, 16), rowd),
                  pl.BlockSpec((2 * F + G, 2 * F), whole),
                  pl.BlockSpec((1, 2 * F), whole)],
        out_specs=[pl.BlockSpec((8, 2 * F), whole),
                   pl.BlockSpec((BE, 2 * F), rowd)],
        out_shape=[jax.ShapeDtypeStruct((8, 2 * F), jnp.float32),
                   jax.ShapeDtypeStruct((EP, 2 * F), jnp.float32)],
    )(g2b, g2b, erec, W, b2)
    return _pass2(t_all, erec, stats, g1, be1, 1.0 / Ee, 4)


def _three_t(gca, trec, Q_ref, b_ref):
    d1 = trec[:, A:A + 1]
    d2 = trec[:, A + 1:A + 2]
    ang = trec[:, 0:A]
    ef1 = _edge_basis(d1)
    ef2 = _edge_basis(d2)
    t = jnp.dot(gca, Q_ref[0:F, :], preferred_element_type=jnp.float32)
    t += jnp.dot(ef1, Q_ref[F:F + G, :], preferred_element_type=jnp.float32)
    t += jnp.dot(ef2, Q_ref[F + G:F + 2 * G, :],
                 preferred_element_type=jnp.float32)
    t += jnp.dot(ang, Q_ref[F + 2 * G:, :], preferred_element_type=jnp.float32)
    return t + b_ref[...]


def _three_pass1_body(gca_ref, tr_ref, Q_ref, b_ref, st_ref, t_ref):
    i = pl.program_id(0)
    t = _three_t(gca_ref[...], tr_ref[...], Q_ref, b_ref)
    t_ref[...] = t
    rid = lax.broadcasted_iota(jnp.int32, (BT, 2 * F), 0) + i * BT
    tm = jnp.where(rid < Tt, t, 0.0)

    @pl.when(i == 0)
    def _():
        st_ref[...] = jnp.zeros_like(st_ref)

    st_ref[0:1, :] += jnp.sum(tm, axis=0, keepdims=True)
    st_ref[1:2, :] += jnp.sum(tm * tm, axis=0, keepdims=True)


def _three_stage(gca, trec, Q, b, g1, be1):
    grid = TP // BT
    row = lambda i: (i, 0)
    whole = lambda i: (0, 0)
    b2 = b.reshape(1, 2 * F)
    K = F + 2 * G + A
    stats, t_all = pl.pallas_call(
        _three_pass1_body,
        grid=(grid,),
        in_specs=[pl.BlockSpec((BT, F), row),
                  pl.BlockSpec((BT, 24), row),
                  pl.BlockSpec((K, 2 * F), whole),
                  pl.BlockSpec((1, 2 * F), whole)],
        out_specs=[pl.BlockSpec((8, 2 * F), whole),
                   pl.BlockSpec((BT, 2 * F), row)],
        out_shape=[jax.ShapeDtypeStruct((8, 2 * F), jnp.float32),
                   jax.ShapeDtypeStruct((TP, 2 * F), jnp.float32)],
    )(gca, trec, Q, b2)
    return _pass2(t_all, trec, stats, g1, be1, 1.0 / Tt, A + 2)


def _node_body(af_ref, ag_ref, g2_ref, be2_ref, out_ref):
    ag = ag_ref[0:Nn, :] + ag_ref[Nn:2 * Nn, :]
    mu = jnp.mean(ag, axis=0, keepdims=True)
    var = jnp.mean(ag * ag, axis=0, keepdims=True) - mu * mu
    an = (ag - mu) / jnp.sqrt(var + 1e-5) * g2_ref[...] + be2_ref[...]
    out_ref[...] = _softplus(af_ref[...] + an)


def _node_update(af, parts, g2, be2):
    return pl.pallas_call(
        _node_body,
        in_specs=[pl.BlockSpec((Nn, F), lambda: (0, 0)),
                  pl.BlockSpec((2 * Nn, F), lambda: (0, 0)),
                  pl.BlockSpec((1, F), lambda: (0, 0)),
                  pl.BlockSpec((1, F), lambda: (0, 0))],
        out_specs=pl.BlockSpec((Nn, F), lambda: (0, 0)),
        out_shape=jax.ShapeDtypeStruct((Nn, F), jnp.float32),
    )(af, parts.reshape(2 * Nn, F), g2.reshape(1, F), be2.reshape(1, F))


def _embed_body(at_ref, emb_ref, out_ref):
    oh = (at_ref[...] == lax.broadcasted_iota(jnp.int32, (1, 128), 1))
    out_ref[...] = jnp.dot(oh.astype(jnp.float32), emb_ref[...],
                           preferred_element_type=jnp.float32)


def _embed(atom_types, emb):
    embp = jnp.concatenate([emb, jnp.zeros((128 - emb.shape[0], F))], axis=0)
    return pl.pallas_call(
        _embed_body,
        in_specs=[pl.BlockSpec((Nn, 1), lambda: (0, 0)),
                  pl.BlockSpec((128, F), lambda: (0, 0))],
        out_specs=pl.BlockSpec((Nn, F), lambda: (0, 0)),
        out_shape=jax.ShapeDtypeStruct((Nn, F), jnp.float32),
    )(atom_types.reshape(Nn, 1).astype(jnp.int32), embp)


def _head_body(af_ref, batch_ref, W1_ref, b1_ref, W2_ref, b2_ref, W3_ref,
               b3_ref, out_ref):
    h = _softplus(jnp.dot(af_ref[...], W1_ref[...],
                          preferred_element_type=jnp.float32) + b1_ref[...])
    h = _softplus(jnp.dot(h, W2_ref[...],
                          preferred_element_type=jnp.float32) + b2_ref[...])
    e = jnp.dot(h, W3_ref[...], preferred_element_type=jnp.float32) \
        + b3_ref[...]
    mask = (batch_ref[...] == lax.broadcasted_iota(jnp.int32, (1, Bb), 1))
    out_ref[...] = jnp.sum(e * mask.astype(jnp.float32), axis=0,
                           keepdims=True)


def _head(af, batch, params_head):
    (W1, b1), (W2, b2), (W3, b3) = params_head
    whole = lambda: (0, 0)
    out = pl.pallas_call(
        _head_body,
        in_specs=[pl.BlockSpec((Nn, F), whole),
                  pl.BlockSpec((Nn, 1), whole),
                  pl.BlockSpec((F, 128), whole),
                  pl.BlockSpec((1, 128), whole),
                  pl.BlockSpec((128, F), whole),
                  pl.BlockSpec((1, F), whole),
                  pl.BlockSpec((F, 1), whole),
                  pl.BlockSpec((1, 1), whole)],
        out_specs=pl.BlockSpec((1, Bb), whole),
        out_shape=jax.ShapeDtypeStruct((1, Bb), jnp.float32),
    )(af, batch.reshape(Nn, 1), W1, b1.reshape(1, 128), W2,
      b2.reshape(1, F), W3, b3.reshape(1, 1))
    return out.reshape(Bb)


def kernel(atom_types, pos, edge_index, edge_offset, triplet_idx, batch,
           num_atoms, volume, params):
    src = edge_index[0].astype(jnp.int32)
    dst = edge_index[1].astype(jnp.int32)
    e1 = triplet_idx[0].astype(jnp.int32)
    e2 = triplet_idx[1].astype(jnp.int32)

    padE = jnp.zeros((EP - Ee,), jnp.int32)
    src_p = jnp.concatenate([src, padE])
    dst_p = jnp.concatenate([dst, padE])
    e1_p = jnp.concatenate([e1, padE])
    e2_p = jnp.concatenate([e2, padE])

    idx_sd = jnp.concatenate([src_p, dst_p]).reshape(NW, 2 * EP // (NW * CH),
                                                     CH)
    idx_e12 = jnp.concatenate([e1_p, e2_p]).reshape(NW, 2 * TP // (NW * CH),
                                                    CH)
    idx_ds = jnp.concatenate([dst_p, src_p]).reshape(NW, 2 * EP // (NW * CH),
                                                     CH)
    idx_dst = dst_p.reshape(NW, EP // (NW * CH), CH)
    off_p = jnp.concatenate([edge_offset,
                             jnp.zeros((EP - Ee, 3), jnp.float32)])
    pos16 = jnp.concatenate([pos, jnp.zeros((Nn, 13), jnp.float32)], axis=1)
    zero_n = jnp.zeros((Nn, F), jnp.float32)

    posg = _sc_gather_small(pos16, idx_sd, 16, 8)
    erec = _edge_record(posg, off_p, dst_p)
    er12 = _sc_gather(erec, idx_e12, 16, 16)
    trec = _triplet_record(er12)
    ca_p = trec[:, A + 3].astype(jnp.int32)
    idx_ca = ca_p.reshape(NW, TP // (NW * CH), CH)

    af = _embed(atom_types, params['embed'])
    for l in range(NC):
        p = params['two'][l]
        g2b = _sc_gather_small(af, idx_ds, F, 4)
        msg = _two_stage(g2b, erec, p['W'], p['b'], p['g1'], p['be1'])
        parts = _sc_scatter_add(msg, idx_dst, zero_n)
        af = _node_update(af, parts, p['g2'], p['be2'])

        q = params['three'][l]
        gca = _sc_gather_small(af, idx_ca, F, 4)
        msg2 = _three_stage(gca, trec, q['W'], q['b'], q['g1'], q['be1'])
        parts2 = _sc_scatter_add(msg2, idx_ca, zero_n)
        af = _node_update(af, parts2, q['g2'], q['be2'])

    return _head(af, batch, params['head'])
